# baseline (speedup 1.0000x reference)
import functools

import jax
import jax.numpy as jnp
from jax import lax
from jax.experimental import pallas as pl
from jax.experimental.pallas import tpu as pltpu
from jax.experimental.pallas import tpu_sc as plsc

DIM = 64
BATCH = 16384
NW = 32
BPW = BATCH // NW
NGROUP = BPW // 16
NPAIR = NGROUP // 2
TLO = 999424


def _stream_impl(idx_list, embT, tail_t, gat, uidx, chunk0, chunk1,
                 wl_u, wl_s, hitstage, hslotv, tailblk, csem0, csem1,
                 *, shift, ch, nfull, kmax, dump):
    wid = lax.axis_index("s") * 2 + lax.axis_index("c")
    lane = lax.iota(jnp.int32, 16)
    zero16i = jnp.zeros((16,), jnp.int32)
    dumpf = jnp.full((16,), float(dump), jnp.float32)
    lane0 = lane == 0

    plsc.store_scatter(hslotv, [zero16i, lane], dumpf)

    cnt = jnp.zeros((16,), jnp.int32)
    for idx2, sbase in idx_list:
        for h in range(2):
            pltpu.sync_copy(idx2.at[pl.ds(h * 64, 64)], uidx)

            def l1(r, cnt, h=h, sbase=sbase):
                for c in range(8):
                    uf = plsc.load_gather(
                        uidx, [jnp.full((16,), 0, jnp.int32) + r,
                               c * 16 + lane])
                    ui = uf.astype(jnp.int32)
                    owner = jnp.minimum(
                        lax.shift_right_logical(ui, shift), nfull)
                    m = owner % 32 == wid
                    mi = m.astype(jnp.int32)
                    pos = cnt + plsc.cumsum(mi) - 1
                    slotf = (sbase + (h * 64 + r) * 128 + c * 16
                             + lane).astype(jnp.float32)
                    plsc.store_scatter(wl_u, [pos], uf, mask=m)
                    plsc.store_scatter(wl_s, [pos], slotf, mask=m)
                    cnt = cnt + plsc.all_reduce_population_count(m)
                return cnt

            cnt = lax.fori_loop(0, 64, l1, cnt)
    n = cnt[0]
    nt = lax.div(n + 15, 16)

    def flush():
        sv = plsc.load_gather(hslotv, [zero16i, lane]).astype(jnp.int32)
        pltpu.sync_copy(hitstage, gat.at[sv])
        plsc.store_scatter(hslotv, [zero16i, lane], dumpf)

    def emit(hc, hrow_vals_fn, slotv):
        hrow = jnp.zeros((16,), jnp.int32) + lax.rem(hc, 16)
        hrow_vals_fn(hrow)
        plsc.store_scatter(hslotv, [zero16i, hrow], slotv, mask=lane0)

        @pl.when(lax.rem(hc, 16) == 15)
        def _():
            flush()

    def emit_hits(t, cid, lo, hc, chunk):
        wuf = plsc.load_gather(wl_u, [t * 16 + lane])
        wu = wuf.astype(jnp.int32)
        wsf = plsc.load_gather(wl_s, [t * 16 + lane])
        valid = (t * 16 + lane) < n
        m = valid & (lax.shift_right_logical(wu, shift) == cid)

        def cond(carry):
            m, hc = carry
            return plsc.all_reduce_population_count(m)[0] > 0

        def body(carry):
            m, hc = carry
            ffs = plsc.all_reduce_ffs(m)
            ulv = wu.at[ffs].get(mode="promise_in_bounds") - lo
            slotv = wsf.at[ffs].get(mode="promise_in_bounds")

            def write(hrow):
                for k4 in range(4):
                    d16 = k4 * 16 + lane
                    vals = plsc.load_gather(chunk, [d16, ulv])
                    plsc.store_scatter(hitstage, [hrow, d16], vals)

            emit(hc, write, slotv)
            m = m & (lane != ffs)
            return m, hc + 1

        m, hc = lax.while_loop(cond, body, (m, hc))
        return hc

    def tail_hits(t, hc):
        wuf = plsc.load_gather(wl_u, [t * 16 + lane])
        wu = wuf.astype(jnp.int32)
        wsf = plsc.load_gather(wl_s, [t * 16 + lane])
        valid = (t * 16 + lane) < n
        m = valid & (wu >= TLO)

        def cond(carry):
            m, hc = carry
            return plsc.all_reduce_population_count(m)[0] > 0

        def body(carry):
            m, hc = carry
            ffs = plsc.all_reduce_ffs(m)
            uv = wu.at[ffs].get(mode="promise_in_bounds") - TLO
            slotv = wsf.at[ffs].get(mode="promise_in_bounds")
            blk = pl.multiple_of((uv[0] // 8) * 8, 8)
            pltpu.sync_copy(tail_t.at[pl.ds(blk, 8)], tailblk)
            srow = lax.rem(uv, 8)

            def write(hrow):
                for k4 in range(4):
                    d16 = k4 * 16 + lane
                    vals = plsc.load_gather(tailblk, [srow, d16])
                    plsc.store_scatter(hitstage, [hrow, d16], vals)

            emit(hc, write, slotv)
            m = m & (lane != ffs)
            return m, hc + 1

        m, hc = lax.while_loop(cond, body, (m, hc))
        return hc

    chunks = (chunk0, chunk1)
    csems = (csem0, csem1)

    def firec(k, buf):
        pltpu.async_copy(embT.at[:, pl.ds((k * 32 + wid) * ch, ch)],
                         chunks[buf], csems[buf])

    def drainc(buf):
        pltpu.make_async_copy(embT.at[:, pl.ds(0, ch)], chunks[buf],
                              csems[buf]).wait()

    def scan_k(hc, k, buf):
        cid = k * 32 + wid
        lo = cid * ch
        drainc(buf)

        def scan_t(t, hc):
            return emit_hits(t, cid, lo, hc, chunks[buf])

        return lax.fori_loop(0, nt, scan_t, hc)

    firec(0, 0)

    def pairloop(kk, hc):
        k0 = 2 * kk
        firec(k0 + 1, 1)
        hc = scan_k(hc, k0, 0)

        @pl.when(k0 + 2 < kmax)
        def _():
            firec(k0 + 2, 0)

        hc = scan_k(hc, k0 + 1, 1)
        return hc

    hc = lax.fori_loop(0, kmax // 2, pairloop, jnp.int32(0))
    if kmax % 2:
        hc = scan_k(hc, kmax - 1, 0)

    def tail(hc):
        return lax.fori_loop(0, nt, tail_hits, hc)

    hc = lax.cond(wid == 0, tail, lambda hc: hc, hc)

    @pl.when(lax.rem(hc, 16) != 0)
    def _():
        flush()


def _k1u_body(users2, embT, tail_t, gat, *rest):
    _stream_impl([(users2, 0)], embT, tail_t, gat, *rest,
                 shift=9, ch=512, nfull=1952, kmax=61, dump=BATCH)


def _k1i_body(pos2, neg2, embT, tail_t, gat, *rest):
    _stream_impl([(pos2, 0), (neg2, BATCH)], embT, tail_t, gat, *rest,
                 shift=8, ch=256, nfull=3904, kmax=122, dump=2 * BATCH)


def _stream_scratch(ch, wl):
    return [
        pltpu.VMEM((64, 128), jnp.float32),
        pltpu.VMEM((DIM, ch), jnp.float32),
        pltpu.VMEM((DIM, ch), jnp.float32),
        pltpu.VMEM((wl,), jnp.float32),
        pltpu.VMEM((wl,), jnp.float32),
        pltpu.VMEM((16, 128), jnp.float32),
        pltpu.VMEM((1, 128), jnp.float32),
        pltpu.VMEM((8, 128), jnp.float32),
        pltpu.SemaphoreType.DMA,
        pltpu.SemaphoreType.DMA,
    ]


_SC_PARAMS = pltpu.CompilerParams(needs_layout_passes=False,
                                  use_tc_tiling_on_sc=True)
_MESH = plsc.VectorSubcoreMesh(core_axis_name="c", subcore_axis_name="s")

_k1u_kernel = functools.partial(
    pl.kernel,
    out_type=jax.ShapeDtypeStruct((BATCH + 1, 128), jnp.float32),
    mesh=_MESH,
    scratch_types=_stream_scratch(512, 16400),
    compiler_params=_SC_PARAMS,
)(_k1u_body)

_k1i_kernel = functools.partial(
    pl.kernel,
    out_type=jax.ShapeDtypeStruct((2 * BATCH + 1, 128), jnp.float32),
    mesh=_MESH,
    scratch_types=_stream_scratch(256, 32800),
    compiler_params=_SC_PARAMS,
)(_k1i_body)


def _k2_body(gat_u, gat_i, x_out, reg_out,
             su0, su1, sp0, sp1, sn0, sn1, x_v, reg_v, sem0, sem1):
    wid = lax.axis_index("s") * 2 + lax.axis_index("c")
    lane = lax.iota(jnp.int32, 16)
    zero16i = jnp.zeros((16,), jnp.int32)

    sems = (sem0, sem1)
    bufs = ((su0, su1), (sp0, sp1), (sn0, sn1))

    def fire(g, buf):
        s = sems[buf]
        base = wid * BPW + g * 16
        pltpu.async_copy(gat_u.at[pl.ds(base, 16)], bufs[0][buf], s)
        pltpu.async_copy(gat_i.at[pl.ds(base, 16)], bufs[1][buf], s)
        pltpu.async_copy(gat_i.at[pl.ds(BATCH + base, 16)], bufs[2][buf], s)

    def drain(buf):
        s = sems[buf]
        for b in bufs:
            pltpu.make_async_copy(gat_u.at[pl.ds(0, 16)], b[buf], s).wait()

    def compute(g, buf, racc):
        sp = jnp.zeros((16,), jnp.float32)
        sn = jnp.zeros((16,), jnp.float32)
        for d in range(DIM):
            cd = jnp.full((16,), d, jnp.int32)
            u = plsc.load_gather(bufs[0][buf], [lane, cd])
            p = plsc.load_gather(bufs[1][buf], [lane, cd])
            n = plsc.load_gather(bufs[2][buf], [lane, cd])
            sp = sp + u * p
            sn = sn + u * n
            racc = racc + (u * u + p * p + n * n)
        xrow = jnp.full((16,), 0, jnp.int32) + lax.div(g, 8)
        xcol = lax.rem(g, 8) * 16 + lane
        plsc.store_scatter(x_v, [xrow, xcol], sp - sn)
        return racc

    fire(0, 0)

    def pair(gg, racc):
        g0 = 2 * gg
        fire(g0 + 1, 1)
        drain(0)
        racc = compute(g0, 0, racc)

        @pl.when(gg < NPAIR - 1)
        def _():
            fire(g0 + 2, 0)

        drain(1)
        racc = compute(g0 + 1, 1, racc)
        return racc

    racc = lax.fori_loop(0, NPAIR, pair, jnp.zeros((16,), jnp.float32))

    plsc.store_scatter(reg_v, [zero16i, lane], racc)
    for k in range(1, 8):
        plsc.store_scatter(reg_v, [zero16i, k * 16 + lane],
                           jnp.zeros((16,), jnp.float32))

    pltpu.sync_copy(x_v, x_out.at[wid])
    pltpu.sync_copy(reg_v, reg_out.at[wid])


_k2_kernel = functools.partial(
    pl.kernel,
    out_type=(
        jax.ShapeDtypeStruct((NW, 4, 128), jnp.float32),
        jax.ShapeDtypeStruct((NW, 1, 128), jnp.float32),
    ),
    mesh=_MESH,
    scratch_types=[
        pltpu.VMEM((16, 128), jnp.float32),
        pltpu.VMEM((16, 128), jnp.float32),
        pltpu.VMEM((16, 128), jnp.float32),
        pltpu.VMEM((16, 128), jnp.float32),
        pltpu.VMEM((16, 128), jnp.float32),
        pltpu.VMEM((16, 128), jnp.float32),
        pltpu.VMEM((4, 128), jnp.float32),
        pltpu.VMEM((1, 128), jnp.float32),
        pltpu.SemaphoreType.DMA,
        pltpu.SemaphoreType.DMA,
    ],
    compiler_params=_SC_PARAMS,
)(_k2_body)


def _tc_body(x_ref, reg_ref, rank_ref, regl_ref):
    x = x_ref[...]
    t = -x
    sp = jnp.maximum(t, 0.0) + jnp.log1p(jnp.exp(-jnp.abs(t)))
    rank_ref[0, 0] = jnp.sum(sp) * (1.0 / BATCH)
    regl_ref[0, 0] = jnp.sum(reg_ref[...]) * (1.0 / BATCH)


_tc_kernel = pl.pallas_call(
    _tc_body,
    out_shape=(
        jax.ShapeDtypeStruct((1, 1), jnp.float32),
        jax.ShapeDtypeStruct((1, 1), jnp.float32),
    ),
    in_specs=[
        pl.BlockSpec(memory_space=pltpu.VMEM),
        pl.BlockSpec(memory_space=pltpu.VMEM),
    ],
    out_specs=(
        pl.BlockSpec(memory_space=pltpu.SMEM),
        pl.BlockSpec(memory_space=pltpu.SMEM),
    ),
)


@jax.jit
def kernel(users, pos_items, neg_items, user_emb, item_emb):
    users2 = users.astype(jnp.float32).reshape(128, 128)
    pos2 = pos_items.astype(jnp.float32).reshape(128, 128)
    neg2 = neg_items.astype(jnp.float32).reshape(128, 128)
    utail = jnp.pad(user_emb[TLO:], ((0, 7), (0, 64)))
    itail = jnp.pad(item_emb[TLO:], ((0, 7), (0, 64)))
    gat_u = _k1u_kernel(users2, user_emb.T, utail)
    gat_i = _k1i_kernel(pos2, neg2, item_emb.T, itail)
    x, reg_part = _k2_kernel(gat_u, gat_i)
    rank, regl = _tc_kernel(x.reshape(128, 128), reg_part.reshape(32, 128))
    return (rank[0, 0], regl[0, 0])


# speedup vs baseline: 1.2416x; 1.2416x over previous
import functools

import jax
import jax.numpy as jnp
from jax import lax
from jax.experimental import pallas as pl
from jax.experimental.pallas import tpu as pltpu
from jax.experimental.pallas import tpu_sc as plsc

DIM = 64
BATCH = 16384
NW = 32
BPW = BATCH // NW
NGROUP = BPW // 16
NPAIR = NGROUP // 2
TLO = 999424


def _stream_impl(idx_list, embT, tail_t, gat, uidx, chunk0, chunk1,
                 wl_u, wl_s, hitstage, hslotv, tailblk, csem0, csem1,
                 *, shift, ch, nfull, kmax, dump, cap):
    wid = lax.axis_index("s") * 2 + lax.axis_index("c")
    lane = lax.iota(jnp.int32, 16)
    zero16i = jnp.zeros((16,), jnp.int32)
    dumpf = jnp.full((16,), float(dump), jnp.float32)
    lane0 = lane == 0

    plsc.store_scatter(hslotv, [zero16i, lane], dumpf)

    def l1_pass(base):
        cnt = jnp.zeros((16,), jnp.int32)
        for idx2, sbase in idx_list:
            for h in range(2):
                pltpu.sync_copy(idx2.at[pl.ds(h * 64, 64)], uidx)

                def l1(r, cnt, h=h, sbase=sbase):
                    for c in range(8):
                        uf = plsc.load_gather(
                            uidx, [jnp.full((16,), 0, jnp.int32) + r,
                                   c * 16 + lane])
                        ui = uf.astype(jnp.int32)
                        owner = jnp.minimum(
                            lax.shift_right_logical(ui, shift), nfull)
                        m = owner % 32 == wid
                        mi = m.astype(jnp.int32)
                        pos = cnt + plsc.cumsum(mi) - 1
                        ms = m & (pos >= base) & (pos < base + cap)
                        slotf = (sbase + (h * 64 + r) * 128 + c * 16
                                 + lane).astype(jnp.float32)
                        plsc.store_scatter(wl_u, [pos - base], uf, mask=ms)
                        plsc.store_scatter(wl_s, [pos - base], slotf,
                                           mask=ms)
                        cnt = cnt + plsc.all_reduce_population_count(m)
                    return cnt

                cnt = lax.fori_loop(0, 64, l1, cnt)
        return cnt

    def flush():
        sv = plsc.load_gather(hslotv, [zero16i, lane]).astype(jnp.int32)
        pltpu.sync_copy(hitstage, gat.at[sv])
        plsc.store_scatter(hslotv, [zero16i, lane], dumpf)

    def emit(hc, hrow_vals_fn, slotv):
        hrow = jnp.zeros((16,), jnp.int32) + lax.rem(hc, 16)
        hrow_vals_fn(hrow)
        plsc.store_scatter(hslotv, [zero16i, hrow], slotv, mask=lane0)

        @pl.when(lax.rem(hc, 16) == 15)
        def _():
            flush()

    def emit_hits(t, cid, lo, hc, chunk, n):
        wuf = plsc.load_gather(wl_u, [t * 16 + lane])
        wu = wuf.astype(jnp.int32)
        wsf = plsc.load_gather(wl_s, [t * 16 + lane])
        valid = (t * 16 + lane) < n
        m = valid & (lax.shift_right_logical(wu, shift) == cid)

        def cond(carry):
            m, hc = carry
            return plsc.all_reduce_population_count(m)[0] > 0

        def body(carry):
            m, hc = carry
            ffs = plsc.all_reduce_ffs(m)
            ulv = wu.at[ffs].get(mode="promise_in_bounds") - lo
            slotv = wsf.at[ffs].get(mode="promise_in_bounds")

            def write(hrow):
                for k4 in range(4):
                    d16 = k4 * 16 + lane
                    vals = plsc.load_gather(chunk, [d16, ulv])
                    plsc.store_scatter(hitstage, [hrow, d16], vals)

            emit(hc, write, slotv)
            m = m & (lane != ffs)
            return m, hc + 1

        m, hc = lax.while_loop(cond, body, (m, hc))
        return hc

    def tail_hits(t, hc, n):
        wuf = plsc.load_gather(wl_u, [t * 16 + lane])
        wu = wuf.astype(jnp.int32)
        wsf = plsc.load_gather(wl_s, [t * 16 + lane])
        valid = (t * 16 + lane) < n
        m = valid & (wu >= TLO)

        def cond(carry):
            m, hc = carry
            return plsc.all_reduce_population_count(m)[0] > 0

        def body(carry):
            m, hc = carry
            ffs = plsc.all_reduce_ffs(m)
            uv = wu.at[ffs].get(mode="promise_in_bounds") - TLO
            slotv = wsf.at[ffs].get(mode="promise_in_bounds")
            blk = pl.multiple_of((uv[0] // 8) * 8, 8)
            pltpu.sync_copy(tail_t.at[pl.ds(blk, 8)], tailblk)
            srow = lax.rem(uv, 8)

            def write(hrow):
                for k4 in range(4):
                    d16 = k4 * 16 + lane
                    vals = plsc.load_gather(tailblk, [srow, d16])
                    plsc.store_scatter(hitstage, [hrow, d16], vals)

            emit(hc, write, slotv)
            m = m & (lane != ffs)
            return m, hc + 1

        m, hc = lax.while_loop(cond, body, (m, hc))
        return hc

    chunks = (chunk0, chunk1)
    csems = (csem0, csem1)

    def firec(k, buf):
        pltpu.async_copy(embT.at[:, pl.ds((k * 32 + wid) * ch, ch)],
                         chunks[buf], csems[buf])

    def drainc(buf):
        pltpu.make_async_copy(embT.at[:, pl.ds(0, ch)], chunks[buf],
                              csems[buf]).wait()

    def run_stream(n, hc):
        nt = lax.div(n + 15, 16)

        def scan_k(hc, k, buf):
            cid = k * 32 + wid
            lo = cid * ch
            drainc(buf)

            def scan_t(t, hc):
                return emit_hits(t, cid, lo, hc, chunks[buf], n)

            return lax.fori_loop(0, nt, scan_t, hc)

        firec(0, 0)

        def pairloop(kk, hc):
            k0 = 2 * kk
            firec(k0 + 1, 1)
            hc = scan_k(hc, k0, 0)

            @pl.when(k0 + 2 < kmax)
            def _():
                firec(k0 + 2, 0)

            hc = scan_k(hc, k0 + 1, 1)
            return hc

        hc = lax.fori_loop(0, kmax // 2, pairloop, hc)
        if kmax % 2:
            hc = scan_k(hc, kmax - 1, 0)

        def tail(hc):
            def tail_t(t, hc):
                return tail_hits(t, hc, n)

            return lax.fori_loop(0, nt, tail_t, hc)

        return lax.cond(wid == 0, tail, lambda hc: hc, hc)

    cnt = l1_pass(0)
    tot = cnt[0]
    hc = run_stream(jnp.minimum(tot, cap), jnp.int32(0))

    def sweep2(hc):
        l1_pass(cap)
        return run_stream(tot - cap, hc)

    hc = lax.cond(tot > cap, sweep2, lambda hc: hc, hc)

    @pl.when(lax.rem(hc, 16) != 0)
    def _():
        flush()


def _k1u_body(users2, embT, tail_t, gat, *rest):
    _stream_impl([(users2, 0)], embT, tail_t, gat, *rest,
                 shift=9, ch=512, nfull=1952, kmax=61, dump=BATCH,
                 cap=BATCH)


def _k1i_body(pos2, neg2, embT, tail_t, gat, *rest):
    _stream_impl([(pos2, 0), (neg2, BATCH)], embT, tail_t, gat, *rest,
                 shift=9, ch=512, nfull=1952, kmax=61, dump=2 * BATCH,
                 cap=BATCH)


def _stream_scratch(ch, wl):
    return [
        pltpu.VMEM((64, 128), jnp.float32),
        pltpu.VMEM((DIM, ch), jnp.float32),
        pltpu.VMEM((DIM, ch), jnp.float32),
        pltpu.VMEM((wl,), jnp.float32),
        pltpu.VMEM((wl,), jnp.float32),
        pltpu.VMEM((16, 128), jnp.float32),
        pltpu.VMEM((1, 128), jnp.float32),
        pltpu.VMEM((8, 128), jnp.float32),
        pltpu.SemaphoreType.DMA,
        pltpu.SemaphoreType.DMA,
    ]


_SC_PARAMS = pltpu.CompilerParams(needs_layout_passes=False,
                                  use_tc_tiling_on_sc=True)
_MESH = plsc.VectorSubcoreMesh(core_axis_name="c", subcore_axis_name="s")

_k1u_kernel = functools.partial(
    pl.kernel,
    out_type=jax.ShapeDtypeStruct((BATCH + 1, 128), jnp.float32),
    mesh=_MESH,
    scratch_types=_stream_scratch(512, 16400),
    compiler_params=_SC_PARAMS,
)(_k1u_body)

_k1i_kernel = functools.partial(
    pl.kernel,
    out_type=jax.ShapeDtypeStruct((2 * BATCH + 1, 128), jnp.float32),
    mesh=_MESH,
    scratch_types=_stream_scratch(512, 16400),
    compiler_params=_SC_PARAMS,
)(_k1i_body)


def _k2_body(gat_u, gat_i, x_out, reg_out,
             su0, su1, sp0, sp1, sn0, sn1, x_v, reg_v, sem0, sem1):
    wid = lax.axis_index("s") * 2 + lax.axis_index("c")
    lane = lax.iota(jnp.int32, 16)
    zero16i = jnp.zeros((16,), jnp.int32)

    sems = (sem0, sem1)
    bufs = ((su0, su1), (sp0, sp1), (sn0, sn1))

    def fire(g, buf):
        s = sems[buf]
        base = wid * BPW + g * 16
        pltpu.async_copy(gat_u.at[pl.ds(base, 16)], bufs[0][buf], s)
        pltpu.async_copy(gat_i.at[pl.ds(base, 16)], bufs[1][buf], s)
        pltpu.async_copy(gat_i.at[pl.ds(BATCH + base, 16)], bufs[2][buf], s)

    def drain(buf):
        s = sems[buf]
        for b in bufs:
            pltpu.make_async_copy(gat_u.at[pl.ds(0, 16)], b[buf], s).wait()

    def compute(g, buf, racc):
        sp = jnp.zeros((16,), jnp.float32)
        sn = jnp.zeros((16,), jnp.float32)
        for d in range(DIM):
            cd = jnp.full((16,), d, jnp.int32)
            u = plsc.load_gather(bufs[0][buf], [lane, cd])
            p = plsc.load_gather(bufs[1][buf], [lane, cd])
            n = plsc.load_gather(bufs[2][buf], [lane, cd])
            sp = sp + u * p
            sn = sn + u * n
            racc = racc + (u * u + p * p + n * n)
        xrow = jnp.full((16,), 0, jnp.int32) + lax.div(g, 8)
        xcol = lax.rem(g, 8) * 16 + lane
        plsc.store_scatter(x_v, [xrow, xcol], sp - sn)
        return racc

    fire(0, 0)

    def pair(gg, racc):
        g0 = 2 * gg
        fire(g0 + 1, 1)
        drain(0)
        racc = compute(g0, 0, racc)

        @pl.when(gg < NPAIR - 1)
        def _():
            fire(g0 + 2, 0)

        drain(1)
        racc = compute(g0 + 1, 1, racc)
        return racc

    racc = lax.fori_loop(0, NPAIR, pair, jnp.zeros((16,), jnp.float32))

    plsc.store_scatter(reg_v, [zero16i, lane], racc)
    for k in range(1, 8):
        plsc.store_scatter(reg_v, [zero16i, k * 16 + lane],
                           jnp.zeros((16,), jnp.float32))

    pltpu.sync_copy(x_v, x_out.at[wid])
    pltpu.sync_copy(reg_v, reg_out.at[wid])


_k2_kernel = functools.partial(
    pl.kernel,
    out_type=(
        jax.ShapeDtypeStruct((NW, 4, 128), jnp.float32),
        jax.ShapeDtypeStruct((NW, 1, 128), jnp.float32),
    ),
    mesh=_MESH,
    scratch_types=[
        pltpu.VMEM((16, 128), jnp.float32),
        pltpu.VMEM((16, 128), jnp.float32),
        pltpu.VMEM((16, 128), jnp.float32),
        pltpu.VMEM((16, 128), jnp.float32),
        pltpu.VMEM((16, 128), jnp.float32),
        pltpu.VMEM((16, 128), jnp.float32),
        pltpu.VMEM((4, 128), jnp.float32),
        pltpu.VMEM((1, 128), jnp.float32),
        pltpu.SemaphoreType.DMA,
        pltpu.SemaphoreType.DMA,
    ],
    compiler_params=_SC_PARAMS,
)(_k2_body)


def _tc_body(x_ref, reg_ref, rank_ref, regl_ref):
    x = x_ref[...]
    t = -x
    sp = jnp.maximum(t, 0.0) + jnp.log1p(jnp.exp(-jnp.abs(t)))
    rank_ref[0, 0] = jnp.sum(sp) * (1.0 / BATCH)
    regl_ref[0, 0] = jnp.sum(reg_ref[...]) * (1.0 / BATCH)


_tc_kernel = pl.pallas_call(
    _tc_body,
    out_shape=(
        jax.ShapeDtypeStruct((1, 1), jnp.float32),
        jax.ShapeDtypeStruct((1, 1), jnp.float32),
    ),
    in_specs=[
        pl.BlockSpec(memory_space=pltpu.VMEM),
        pl.BlockSpec(memory_space=pltpu.VMEM),
    ],
    out_specs=(
        pl.BlockSpec(memory_space=pltpu.SMEM),
        pl.BlockSpec(memory_space=pltpu.SMEM),
    ),
)


@jax.jit
def kernel(users, pos_items, neg_items, user_emb, item_emb):
    users2 = users.astype(jnp.float32).reshape(128, 128)
    pos2 = pos_items.astype(jnp.float32).reshape(128, 128)
    neg2 = neg_items.astype(jnp.float32).reshape(128, 128)
    utail = jnp.pad(user_emb[TLO:], ((0, 7), (0, 64)))
    itail = jnp.pad(item_emb[TLO:], ((0, 7), (0, 64)))
    gat_u = _k1u_kernel(users2, user_emb.T, utail)
    gat_i = _k1i_kernel(pos2, neg2, item_emb.T, itail)
    x, reg_part = _k2_kernel(gat_u, gat_i)
    rank, regl = _tc_kernel(x.reshape(128, 128), reg_part.reshape(32, 128))
    return (rank[0, 0], regl[0, 0])


# speedup vs baseline: 1.3843x; 1.1149x over previous
import functools

import jax
import jax.numpy as jnp
from jax import lax
from jax.experimental import pallas as pl
from jax.experimental.pallas import tpu as pltpu
from jax.experimental.pallas import tpu_sc as plsc

DIM = 64
BATCH = 16384
NW = 32
BPW = BATCH // NW
NGROUP = BPW // 16
NPAIR = NGROUP // 2
NBLK = 125000
CH = 512
KMAX = 61
DUMP = BATCH


NFULL = 1952
TLO = NFULL * CH


def _k1_body(users2, uembT, utail, gat_u, uidx, chunk0, chunk1, wl_u, wl_s,
             hitstage, hslotv, tailblk, csem0, csem1):
    wid = lax.axis_index("s") * 2 + lax.axis_index("c")
    lane = lax.iota(jnp.int32, 16)
    zero16i = jnp.zeros((16,), jnp.int32)
    dumpf = jnp.full((16,), float(DUMP), jnp.float32)
    lane0 = lane == 0

    plsc.store_scatter(hslotv, [zero16i, lane], dumpf)

    cnt = jnp.zeros((16,), jnp.int32)
    for h in range(2):
        pltpu.sync_copy(users2.at[pl.ds(h * 64, 64)], uidx)

        def l1(r, cnt, h=h):
            for c in range(8):
                uf = plsc.load_gather(uidx,
                                      [jnp.full((16,), 0, jnp.int32) + r,
                                       c * 16 + lane])
                ui = uf.astype(jnp.int32)
                owner = jnp.minimum(lax.shift_right_logical(ui, 9), NFULL)
                m = owner % 32 == wid
                mi = m.astype(jnp.int32)
                pos = cnt + plsc.cumsum(mi) - 1
                slotf = ((h * 64 + r) * 128 + c * 16 + lane).astype(
                    jnp.float32)
                plsc.store_scatter(wl_u, [pos], uf, mask=m)
                plsc.store_scatter(wl_s, [pos], slotf, mask=m)
                cnt = cnt + plsc.all_reduce_population_count(m)
            return cnt

        cnt = lax.fori_loop(0, 64, l1, cnt)
    n = cnt[0]
    nt = lax.div(n + 15, 16)

    def flush():
        sv = plsc.load_gather(hslotv, [zero16i, lane]).astype(jnp.int32)
        pltpu.sync_copy(hitstage, gat_u.at[sv])
        plsc.store_scatter(hslotv, [zero16i, lane], dumpf)

    def emit_hits(t, cid, lo, hc, chunk):
        wuf = plsc.load_gather(wl_u, [t * 16 + lane])
        wu = wuf.astype(jnp.int32)
        wsf = plsc.load_gather(wl_s, [t * 16 + lane])
        valid = (t * 16 + lane) < n
        m = valid & (lax.shift_right_logical(wu, 9) == cid)

        def cond(carry):
            m, hc = carry
            return plsc.all_reduce_population_count(m)[0] > 0

        def body(carry):
            m, hc = carry
            ffs = plsc.all_reduce_ffs(m)
            ulv = wu.at[ffs].get(mode="promise_in_bounds") - lo
            slotv = wsf.at[ffs].get(mode="promise_in_bounds")
            hrow = jnp.zeros((16,), jnp.int32) + lax.rem(hc, 16)
            for k4 in range(4):
                d16 = k4 * 16 + lane
                vals = plsc.load_gather(chunk, [d16, ulv])
                plsc.store_scatter(hitstage, [hrow, d16], vals)
            plsc.store_scatter(hslotv, [zero16i, hrow], slotv, mask=lane0)

            @pl.when(lax.rem(hc, 16) == 15)
            def _():
                flush()

            m = m & (lane != ffs)
            return m, hc + 1

        m, hc = lax.while_loop(cond, body, (m, hc))
        return hc

    def tail_hits(t, hc):
        wuf = plsc.load_gather(wl_u, [t * 16 + lane])
        wu = wuf.astype(jnp.int32)
        wsf = plsc.load_gather(wl_s, [t * 16 + lane])
        valid = (t * 16 + lane) < n
        m = valid & (wu >= TLO)

        def cond(carry):
            m, hc = carry
            return plsc.all_reduce_population_count(m)[0] > 0

        def body(carry):
            m, hc = carry
            ffs = plsc.all_reduce_ffs(m)
            uv = wu.at[ffs].get(mode="promise_in_bounds") - TLO
            slotv = wsf.at[ffs].get(mode="promise_in_bounds")
            blk = pl.multiple_of((uv[0] // 8) * 8, 8)
            pltpu.sync_copy(utail.at[pl.ds(blk, 8)], tailblk)
            srow = lax.rem(uv, 8)
            hrow = jnp.zeros((16,), jnp.int32) + lax.rem(hc, 16)
            for k4 in range(4):
                d16 = k4 * 16 + lane
                vals = plsc.load_gather(tailblk, [srow, d16])
                plsc.store_scatter(hitstage, [hrow, d16], vals)
            plsc.store_scatter(hslotv, [zero16i, hrow], slotv, mask=lane0)

            @pl.when(lax.rem(hc, 16) == 15)
            def _():
                flush()

            m = m & (lane != ffs)
            return m, hc + 1

        m, hc = lax.while_loop(cond, body, (m, hc))
        return hc

    chunks = (chunk0, chunk1)
    csems = (csem0, csem1)

    def firec(k, buf):
        pltpu.async_copy(uembT.at[:, pl.ds((k * 32 + wid) * CH, CH)],
                         chunks[buf], csems[buf])

    def drainc(buf):
        pltpu.make_async_copy(uembT.at[:, pl.ds(0, CH)], chunks[buf],
                              csems[buf]).wait()

    def scan_k(hc, k, buf):
        cid = k * 32 + wid
        lo = cid * CH
        drainc(buf)

        def scan_t(t, hc):
            return emit_hits(t, cid, lo, hc, chunks[buf])

        return lax.fori_loop(0, nt, scan_t, hc)

    firec(0, 0)

    def pairloop(kk, hc):
        k0 = 2 * kk
        firec(k0 + 1, 1)
        hc = scan_k(hc, k0, 0)
        firec(k0 + 2, 0)
        hc = scan_k(hc, k0 + 1, 1)
        return hc

    hc = lax.fori_loop(0, (KMAX - 1) // 2, pairloop, jnp.int32(0))
    hc = scan_k(hc, KMAX - 1, 0)

    def tail(hc):
        return lax.fori_loop(0, nt, tail_hits, hc)

    hc = lax.cond(wid == 0, tail, lambda hc: hc, hc)

    @pl.when(lax.rem(hc, 16) != 0)
    def _():
        flush()


_k1_kernel = functools.partial(
    pl.kernel,
    out_type=jax.ShapeDtypeStruct((BATCH + 1, 128), jnp.float32),
    mesh=plsc.VectorSubcoreMesh(core_axis_name="c", subcore_axis_name="s"),
    scratch_types=[
        pltpu.VMEM((64, 128), jnp.float32),
        pltpu.VMEM((DIM, CH), jnp.float32),
        pltpu.VMEM((DIM, CH), jnp.float32),
        pltpu.VMEM((16400,), jnp.float32),
        pltpu.VMEM((16400,), jnp.float32),
        pltpu.VMEM((16, 128), jnp.float32),
        pltpu.VMEM((1, 128), jnp.float32),
        pltpu.VMEM((8, 128), jnp.float32),
        pltpu.SemaphoreType.DMA,
        pltpu.SemaphoreType.DMA,
    ],
    compiler_params=pltpu.CompilerParams(needs_layout_passes=False,
                                         use_tc_tiling_on_sc=True),
)(_k1_body)


def _k2_body(pos3, neg3, iemb3, gat_u, x_out, reg_out,
             vid_p, vid_n, su0, su1,
             sp0, sp1, sn0, sn1, x_v, reg_v, sem0, sem1):
    wid = lax.axis_index("s") * 2 + lax.axis_index("c")
    lane = lax.iota(jnp.int32, 16)
    zero16i = jnp.zeros((16,), jnp.int32)

    pltpu.sync_copy(pos3.at[wid], vid_p)
    pltpu.sync_copy(neg3.at[wid], vid_n)

    sems = (sem0, sem1)
    ustages = (su0, su1)
    tabs = ((vid_p, (sp0, sp1)), (vid_n, (sn0, sn1)))

    def gidx(g, vid):
        irow = jnp.full((16,), 0, jnp.int32) + lax.div(g * 16, 128)
        icol = lax.rem(g * 16, 128) + lane
        return plsc.load_gather(vid, [irow, icol]).astype(jnp.int32)

    def fire(g, buf):
        s = sems[buf]
        pltpu.async_copy(gat_u.at[pl.ds(wid * BPW + g * 16, 16)],
                         ustages[buf], s)
        for vid, stages in tabs:
            bv = lax.div(gidx(g, vid), 8)
            for i in range(16):
                pltpu.async_copy(iemb3.at[bv[i]], stages[buf].at[i], s)

    def drain(buf):
        s = sems[buf]
        pltpu.make_async_copy(gat_u.at[pl.ds(0, 16)], ustages[buf],
                              s).wait()
        for vid, stages in tabs:
            pltpu.make_async_copy(iemb3.at[pl.ds(0, 16)], stages[buf],
                                  s).wait()

    def compute(g, buf, racc):
        svp = lax.rem(gidx(g, vid_p), 8)
        svn = lax.rem(gidx(g, vid_n), 8)
        sp = jnp.zeros((16,), jnp.float32)
        sn = jnp.zeros((16,), jnp.float32)
        for d in range(DIM):
            cd = jnp.full((16,), d, jnp.int32)
            u = plsc.load_gather(su0 if buf == 0 else su1, [lane, cd])
            p = plsc.load_gather(sp0 if buf == 0 else sp1, [lane, svp, cd])
            n = plsc.load_gather(sn0 if buf == 0 else sn1, [lane, svn, cd])
            sp = sp + u * p
            sn = sn + u * n
            racc = racc + (u * u + p * p + n * n)
        xrow = jnp.full((16,), 0, jnp.int32) + lax.div(g, 8)
        xcol = lax.rem(g, 8) * 16 + lane
        plsc.store_scatter(x_v, [xrow, xcol], sp - sn)
        return racc

    fire(0, 0)

    def pair(gg, racc):
        g0 = 2 * gg
        fire(g0 + 1, 1)
        drain(0)
        racc = compute(g0, 0, racc)

        @pl.when(gg < NPAIR - 1)
        def _():
            fire(g0 + 2, 0)

        drain(1)
        racc = compute(g0 + 1, 1, racc)
        return racc

    racc = lax.fori_loop(0, NPAIR, pair, jnp.zeros((16,), jnp.float32))

    plsc.store_scatter(reg_v, [zero16i, lane], racc)
    for k in range(1, 8):
        plsc.store_scatter(reg_v, [zero16i, k * 16 + lane],
                           jnp.zeros((16,), jnp.float32))

    pltpu.sync_copy(x_v, x_out.at[wid])
    pltpu.sync_copy(reg_v, reg_out.at[wid])


_k2_kernel = functools.partial(
    pl.kernel,
    out_type=(
        jax.ShapeDtypeStruct((NW, 4, 128), jnp.float32),
        jax.ShapeDtypeStruct((NW, 1, 128), jnp.float32),
    ),
    mesh=plsc.VectorSubcoreMesh(core_axis_name="c", subcore_axis_name="s"),
    scratch_types=[
        pltpu.VMEM((4, 128), jnp.float32),
        pltpu.VMEM((4, 128), jnp.float32),
        pltpu.VMEM((16, 128), jnp.float32),
        pltpu.VMEM((16, 128), jnp.float32),
        pltpu.VMEM((16, 8, DIM), jnp.float32),
        pltpu.VMEM((16, 8, DIM), jnp.float32),
        pltpu.VMEM((16, 8, DIM), jnp.float32),
        pltpu.VMEM((16, 8, DIM), jnp.float32),
        pltpu.VMEM((4, 128), jnp.float32),
        pltpu.VMEM((1, 128), jnp.float32),
        pltpu.SemaphoreType.DMA,
        pltpu.SemaphoreType.DMA,
    ],
    compiler_params=pltpu.CompilerParams(needs_layout_passes=False,
                                         use_tc_tiling_on_sc=True),
)(_k2_body)


def _tc_body(x_ref, reg_ref, rank_ref, regl_ref):
    x = x_ref[...]
    t = -x
    sp = jnp.maximum(t, 0.0) + jnp.log1p(jnp.exp(-jnp.abs(t)))
    rank_ref[0, 0] = jnp.sum(sp) * (1.0 / BATCH)
    regl_ref[0, 0] = jnp.sum(reg_ref[...]) * (1.0 / BATCH)


_tc_kernel = pl.pallas_call(
    _tc_body,
    out_shape=(
        jax.ShapeDtypeStruct((1, 1), jnp.float32),
        jax.ShapeDtypeStruct((1, 1), jnp.float32),
    ),
    in_specs=[
        pl.BlockSpec(memory_space=pltpu.VMEM),
        pl.BlockSpec(memory_space=pltpu.VMEM),
    ],
    out_specs=(
        pl.BlockSpec(memory_space=pltpu.SMEM),
        pl.BlockSpec(memory_space=pltpu.SMEM),
    ),
)


@jax.jit
def kernel(users, pos_items, neg_items, user_emb, item_emb):
    users2 = users.astype(jnp.float32).reshape(128, 128)
    pos3 = pos_items.astype(jnp.float32).reshape(NW, 4, 128)
    neg3 = neg_items.astype(jnp.float32).reshape(NW, 4, 128)
    iemb3 = item_emb[:NBLK * 8].reshape(NBLK, 8, DIM)
    utail = jnp.pad(user_emb[TLO:], ((0, 7), (0, 64)))
    gat_u = _k1_kernel(users2, user_emb.T, utail)
    x, reg_part = _k2_kernel(pos3, neg3, iemb3, gat_u)
    rank, regl = _tc_kernel(x.reshape(128, 128), reg_part.reshape(32, 128))
    return (rank[0, 0], regl[0, 0])
